# baseline (speedup 1.0000x reference)
import functools

import jax
import jax.numpy as jnp
from jax import lax
from jax.experimental import pallas as pl
from jax.experimental.pallas import tpu as pltpu
from jax.experimental.pallas import tpu_sc as plsc

B, H, W, C = 2, 512, 512, 96
P = B * H * W
LOG2_W = 9
LOG2_HW = 18

NC = 2
NS = 16
NW = NC * NS
PPW = P // NW
K = 128
NCHUNK = PPW // K
L = 16
NV = C // L


def _splat(vec, j):
    return vec.at[jnp.full((L,), j, jnp.int32)].get(mode="promise_in_bounds")


def _warp_body(img_hbm, flow_hbm, out_hbm,
               flow_v, idx_v, ax_v, ay_v, corners_v, out_v, sem):
    wid = lax.axis_index("s") * NC + lax.axis_index("c")
    lanes = lax.broadcasted_iota(jnp.int32, (L,), 0)
    idx_even = (2 * lanes) & (L - 1)
    idx_odd = (2 * lanes + 1) & (L - 1)
    low_half = lanes < (L // 2)

    def chunk_body(ci, _):
        base = wid * PPW + ci * K

        pltpu.sync_copy(flow_hbm.at[pl.ds(base * 2, K * 2)], flow_v)

        for g in range(K // L):
            rows = g * L + lanes
            p = base + rows
            x = p & (W - 1)
            y = (p >> LOG2_W) & (H - 1)
            bbase = (p >> LOG2_HW) << LOG2_HW

            va = flow_v[pl.ds(g * 2 * L, L)]
            vb = flow_v[pl.ds(g * 2 * L + L, L)]
            fl_y = jnp.where(low_half,
                             va.at[idx_even].get(mode="promise_in_bounds"),
                             vb.at[idx_even].get(mode="promise_in_bounds"))
            fl_x = jnp.where(low_half,
                             va.at[idx_odd].get(mode="promise_in_bounds"),
                             vb.at[idx_odd].get(mode="promise_in_bounds"))

            qy = jnp.clip(y.astype(jnp.float32) - fl_y, 0.0, float(H - 1))
            qx = jnp.clip(x.astype(jnp.float32) - fl_x, 0.0, float(W - 1))
            fy = jnp.minimum(qy.astype(jnp.int32), H - 2)
            fx = jnp.minimum(qx.astype(jnp.int32), W - 2)
            ay = jnp.clip(qy - fy.astype(jnp.float32), 0.0, 1.0)
            ax = jnp.clip(qx - fx.astype(jnp.float32), 0.0, 1.0)

            sl = pl.ds(g * L, L)
            rtl = bbase + (fy << LOG2_W) + fx
            idx_v[0, sl] = rtl
            idx_v[1, sl] = rtl + 1
            idx_v[2, sl] = rtl + W
            idx_v[3, sl] = rtl + W + 1
            ax_v[sl] = ax
            ay_v[sl] = ay

        cps = [pltpu.async_copy(img_hbm.at[idx_v.at[j]], corners_v.at[j], sem)
               for j in range(4)]
        for cp in cps:
            cp.wait()

        def blend(g, _):
            ax16 = ax_v[pl.ds(g * L, L)]
            ay16 = ay_v[pl.ds(g * L, L)]
            for j in range(L):
                i = g * L + j
                axs = _splat(ax16, j)
                ays = _splat(ay16, j)
                for v in range(NV):
                    csl = pl.ds(v * L, L)
                    tl = corners_v[0, i, csl]
                    tr = corners_v[1, i, csl]
                    bl = corners_v[2, i, csl]
                    br = corners_v[3, i, csl]
                    top = tl + axs * (tr - tl)
                    bot = bl + axs * (br - bl)
                    out_v[i, csl] = top + ays * (bot - top)
            return 0

        lax.fori_loop(0, K // L, blend, 0)

        pltpu.sync_copy(out_v, out_hbm.at[pl.ds(base, K)])
        return 0

    lax.fori_loop(0, NCHUNK, chunk_body, 0)


@jax.jit
def _warp(img_flat, flow_flat):
    f = pl.kernel(
        _warp_body,
        out_type=jax.ShapeDtypeStruct((P, C), jnp.float32),
        mesh=plsc.VectorSubcoreMesh(core_axis_name="c", subcore_axis_name="s"),
        compiler_params=pltpu.CompilerParams(use_tc_tiling_on_sc=False),
        scratch_types=[
            pltpu.VMEM((K * 2,), jnp.float32),
            pltpu.VMEM((4, K), jnp.int32),
            pltpu.VMEM((K,), jnp.float32),
            pltpu.VMEM((K,), jnp.float32),
            pltpu.VMEM((4, K, C), jnp.float32),
            pltpu.VMEM((K, C), jnp.float32),
            pltpu.SemaphoreType.DMA,
        ],
    )
    return f(img_flat, flow_flat)


def kernel(image, flow):
    img_flat = image.reshape(P, C)
    flow_flat = flow.reshape(P * 2)
    out = _warp(img_flat, flow_flat)
    return out.reshape(B, H, W, C)


# speedup vs baseline: 1.1484x; 1.1484x over previous
import functools

import jax
import jax.numpy as jnp
from jax import lax
from jax.experimental import pallas as pl
from jax.experimental.pallas import tpu as pltpu
from jax.experimental.pallas import tpu_sc as plsc

B, H, W, C = 2, 512, 512, 96
P = B * H * W
LOG2_W = 9
LOG2_HW = 18

NC = 2
NS = 16
NW = NC * NS
PPW = P // NW
K = 128
NCHUNK = PPW // K
L = 16
NV = C // L
CP = 128


def _splat(vec, j):
    return vec.at[jnp.full((L,), j, jnp.int32)].get(mode="promise_in_bounds")


def _warp_body(img_hbm, flow_hbm, out_hbm,
               flow_v, idx_v, ax_v, ay_v, corners_v, out_v, sem):
    wid = lax.axis_index("s") * NC + lax.axis_index("c")
    lanes = lax.broadcasted_iota(jnp.int32, (L,), 0)
    idx_even = (2 * lanes) & (L - 1)
    idx_odd = (2 * lanes + 1) & (L - 1)
    low_half = lanes < (L // 2)

    def chunk_body(ci, _):
        base = wid * PPW + ci * K

        pltpu.sync_copy(flow_hbm.at[pl.ds(base * 2, K * 2)], flow_v)

        for g in range(K // L):
            rows = g * L + lanes
            p = base + rows
            x = p & (W - 1)
            y = (p >> LOG2_W) & (H - 1)
            bbase = (p >> LOG2_HW) << LOG2_HW

            va = flow_v[pl.ds(g * 2 * L, L)]
            vb = flow_v[pl.ds(g * 2 * L + L, L)]
            fl_y = jnp.where(low_half,
                             va.at[idx_even].get(mode="promise_in_bounds"),
                             vb.at[idx_even].get(mode="promise_in_bounds"))
            fl_x = jnp.where(low_half,
                             va.at[idx_odd].get(mode="promise_in_bounds"),
                             vb.at[idx_odd].get(mode="promise_in_bounds"))

            qy = jnp.clip(y.astype(jnp.float32) - fl_y, 0.0, float(H - 1))
            qx = jnp.clip(x.astype(jnp.float32) - fl_x, 0.0, float(W - 1))
            fy = jnp.minimum(qy.astype(jnp.int32), H - 2)
            fx = jnp.minimum(qx.astype(jnp.int32), W - 2)
            ay = jnp.clip(qy - fy.astype(jnp.float32), 0.0, 1.0)
            ax = jnp.clip(qx - fx.astype(jnp.float32), 0.0, 1.0)

            sl = pl.ds(g * L, L)
            rtl = bbase + (fy << LOG2_W) + fx
            idx_v[0, sl] = rtl
            idx_v[1, sl] = rtl + 1
            idx_v[2, sl] = rtl + W
            idx_v[3, sl] = rtl + W + 1
            ax_v[sl] = ax
            ay_v[sl] = ay

        cps = [pltpu.async_copy(img_hbm.at[idx_v.at[j]], corners_v.at[j], sem)
               for j in range(4)]
        for cp in cps:
            cp.wait()

        def blend(g, _):
            ax16 = ax_v[pl.ds(g * L, L)]
            ay16 = ay_v[pl.ds(g * L, L)]
            for j in range(L):
                i = g * L + j
                axs = _splat(ax16, j)
                ays = _splat(ay16, j)
                for v in range(NV):
                    csl = pl.ds(v * L, L)
                    tl = corners_v[0, i, csl]
                    tr = corners_v[1, i, csl]
                    bl = corners_v[2, i, csl]
                    br = corners_v[3, i, csl]
                    top = tl + axs * (tr - tl)
                    bot = bl + axs * (br - bl)
                    out_v[i, csl] = top + ays * (bot - top)
            return 0

        lax.fori_loop(0, K // L, blend, 0)

        pltpu.sync_copy(out_v, out_hbm.at[pl.ds(base, K)])
        return 0

    lax.fori_loop(0, NCHUNK, chunk_body, 0)


@jax.jit
def _warp(img_flat, flow_flat):
    f = pl.kernel(
        _warp_body,
        out_type=jax.ShapeDtypeStruct((P, C), jnp.float32),
        mesh=plsc.VectorSubcoreMesh(core_axis_name="c", subcore_axis_name="s"),
        compiler_params=pltpu.CompilerParams(use_tc_tiling_on_sc=True),
        scratch_types=[
            pltpu.VMEM((K * 2,), jnp.float32),
            pltpu.VMEM((4, K), jnp.int32),
            pltpu.VMEM((K,), jnp.float32),
            pltpu.VMEM((K,), jnp.float32),
            pltpu.VMEM((4, K, CP), jnp.float32),
            pltpu.VMEM((K, C), jnp.float32),
            pltpu.SemaphoreType.DMA,
        ],
    )
    return f(img_flat, flow_flat)


def kernel(image, flow):
    img_pad = jnp.pad(image.reshape(P, C), ((0, 0), (0, CP - C)))
    flow_flat = flow.reshape(P * 2)
    out = _warp(img_pad, flow_flat)
    return out.reshape(B, H, W, C)


# speedup vs baseline: 1.3730x; 1.1956x over previous
import jax
import jax.numpy as jnp
from jax import lax
from jax.experimental import pallas as pl
from jax.experimental.pallas import tpu as pltpu
from jax.experimental.pallas import tpu_sc as plsc

B, H, W, C = 2, 512, 512, 96
P = B * H * W
LOG2_W = 9
LOG2_HW = 18

NC = 2
NS = 16
NW = NC * NS
PPW = P // NW
K = 64
NCHUNK = PPW // K
L = 16
NV = C // L
CP = 128
NB = 2


def _splat(vec, j):
    return vec.at[jnp.full((L,), j, jnp.int32)].get(mode="promise_in_bounds")


def _warp_body(img_hbm, flow_hbm, out_hbm,
               flow_v, idx_v, ax_v, ay_v, corners_v, out_v,
               gsem, fsem, osem):
    wid = lax.axis_index("s") * NC + lax.axis_index("c")
    lanes = lax.broadcasted_iota(jnp.int32, (L,), 0)
    idx_even = (2 * lanes) & (L - 1)
    idx_odd = (2 * lanes + 1) & (L - 1)
    low_half = lanes < (L // 2)
    wbase = wid * PPW

    def flow_start(ci, b):
        base = wbase + jnp.minimum(ci, NCHUNK - 1) * K
        pltpu.async_copy(flow_hbm.at[pl.ds(base * 2, K * 2)], flow_v.at[b],
                         fsem)

    def flow_wait(b):
        pltpu.make_async_copy(flow_hbm.at[pl.ds(0, K * 2)], flow_v.at[b],
                              fsem).wait()

    def compute_idx(ci, b):
        base = wbase + jnp.minimum(ci, NCHUNK - 1) * K
        for g in range(K // L):
            rows = g * L + lanes
            p = base + rows
            x = p & (W - 1)
            y = (p >> LOG2_W) & (H - 1)
            bbase = (p >> LOG2_HW) << LOG2_HW

            va = flow_v[b, pl.ds(g * 2 * L, L)]
            vb = flow_v[b, pl.ds(g * 2 * L + L, L)]
            fl_y = jnp.where(low_half,
                             va.at[idx_even].get(mode="promise_in_bounds"),
                             vb.at[idx_even].get(mode="promise_in_bounds"))
            fl_x = jnp.where(low_half,
                             va.at[idx_odd].get(mode="promise_in_bounds"),
                             vb.at[idx_odd].get(mode="promise_in_bounds"))

            qy = jnp.clip(y.astype(jnp.float32) - fl_y, 0.0, float(H - 1))
            qx = jnp.clip(x.astype(jnp.float32) - fl_x, 0.0, float(W - 1))
            fy = jnp.minimum(qy.astype(jnp.int32), H - 2)
            fx = jnp.minimum(qx.astype(jnp.int32), W - 2)
            ay = jnp.clip(qy - fy.astype(jnp.float32), 0.0, 1.0)
            ax = jnp.clip(qx - fx.astype(jnp.float32), 0.0, 1.0)

            sl = pl.ds(g * L, L)
            rtl = bbase + (fy << LOG2_W) + fx
            idx_v[b, 0, sl] = rtl
            idx_v[b, 1, sl] = rtl + 1
            idx_v[b, 2, sl] = rtl + W
            idx_v[b, 3, sl] = rtl + W + 1
            ax_v[b, sl] = ax
            ay_v[b, sl] = ay

    def gather_start(b):
        for j in range(4):
            pltpu.async_copy(img_hbm.at[idx_v.at[b, j]], corners_v.at[b, j],
                             gsem)

    def gather_wait(b):
        for j in range(4):
            pltpu.make_async_copy(img_hbm.at[idx_v.at[b, j]],
                                  corners_v.at[b, j], gsem).wait()

    def blend(b):
        def gbody(g, _):
            ax16 = ax_v[b, pl.ds(g * L, L)]
            ay16 = ay_v[b, pl.ds(g * L, L)]
            for j in range(L):
                i = g * L + j
                axs = _splat(ax16, j)
                ays = _splat(ay16, j)
                for v in range(NV):
                    csl = pl.ds(v * L, L)
                    tl = corners_v[b, 0, i, csl]
                    tr = corners_v[b, 1, i, csl]
                    bl = corners_v[b, 2, i, csl]
                    br = corners_v[b, 3, i, csl]
                    top = tl + axs * (tr - tl)
                    bot = bl + axs * (br - bl)
                    out_v[b, i, csl] = top + ays * (bot - top)
            return 0

        lax.fori_loop(0, K // L, gbody, 0)

    def out_start(ci, b):
        base = wbase + ci * K
        pltpu.async_copy(out_v.at[b], out_hbm.at[pl.ds(base, K)], osem)

    def out_wait(b):
        pltpu.make_async_copy(out_v.at[b], out_hbm.at[pl.ds(0, K)],
                              osem).wait()

    def chunk_step(ci, b, nb, reuse_out):
        flow_wait(nb)
        compute_idx(ci + 1, nb)
        gather_wait(b)
        gather_start(nb)
        flow_start(ci + 2, b)

        @pl.when(reuse_out)
        def _():
            out_wait(b)

        blend(b)
        out_start(ci, b)

    flow_start(0, 0)
    flow_start(1, 1)
    flow_wait(0)
    compute_idx(0, 0)
    gather_start(0)

    def steady_body(half, _):
        ci = half * NB
        chunk_step(ci, 0, 1, half >= 1)
        chunk_step(ci + 1, 1, 0, half >= 1)
        return 0

    lax.fori_loop(0, NCHUNK // NB, steady_body, 0)

    gather_wait(0)
    flow_wait(1)
    out_wait(0)
    out_wait(1)


@jax.jit
def _warp(img_pad, flow_flat):
    f = pl.kernel(
        _warp_body,
        out_type=jax.ShapeDtypeStruct((P, C), jnp.float32),
        mesh=plsc.VectorSubcoreMesh(core_axis_name="c", subcore_axis_name="s"),
        compiler_params=pltpu.CompilerParams(use_tc_tiling_on_sc=True),
        scratch_types=[
            pltpu.VMEM((NB, K * 2), jnp.float32),
            pltpu.VMEM((NB, 4, K), jnp.int32),
            pltpu.VMEM((NB, K), jnp.float32),
            pltpu.VMEM((NB, K), jnp.float32),
            pltpu.VMEM((NB, 4, K, CP), jnp.float32),
            pltpu.VMEM((NB, K, C), jnp.float32),
            pltpu.SemaphoreType.DMA,
            pltpu.SemaphoreType.DMA,
            pltpu.SemaphoreType.DMA,
        ],
    )
    return f(img_pad, flow_flat)


def kernel(image, flow):
    img_pad = jnp.pad(image.reshape(P, C), ((0, 0), (0, CP - C)))
    flow_flat = flow.reshape(P * 2)
    out = _warp(img_pad, flow_flat)
    return out.reshape(B, H, W, C)
